# Initial kernel scaffold; baseline (speedup 1.0000x reference)
import jax
import jax.numpy as jnp
from jax.experimental import pallas as pl


def kernel(text, topic, table):
    raise NotImplementedError("write your pallas kernel here")


# speedup vs baseline: 3.7357x; 3.7357x over previous
import functools

import jax
import jax.numpy as jnp
from jax import lax
from jax.experimental import pallas as pl
from jax.experimental.pallas import tpu as pltpu
from jax.experimental.pallas import tpu_sc as plsc

_VOCAB = 100000
_DIM = 64
_B = 4096
_L_TXT = 200
_L_TOP = 20
_N_TXT = _B * _L_TXT
_N_TOP = _B * _L_TOP

_NC = 2
_NS = 16
_NW = _NC * _NS

_CL = 128
_K = 4
_C = _K * _CL

_TXT_ROWS_W = _N_TXT // _NW // _CL
_TOP_ROWS_W = _N_TOP // _NW // _CL
_TXT_CHUNKS = _TXT_ROWS_W // _K
_TOP_CHUNKS = _TOP_ROWS_W // _K


def _gather_stripe(idx_hbm, out_hbm, table_hbm, idx_v, rows_v, sem,
                   base_blk, n_chunks):
  def body(g, carry):
    blk0 = base_blk + g * _K
    pltpu.sync_copy(idx_hbm.at[pl.ds(blk0, _K)], idx_v)
    copies = [
        pltpu.async_copy(table_hbm.at[idx_v.at[j]], rows_v.at[j], sem)
        for j in range(_K)
    ]
    for cp in copies:
      cp.wait()
    pltpu.sync_copy(rows_v, out_hbm.at[pl.ds(blk0, _K)])
    return carry

  lax.fori_loop(0, n_chunks, body, 0)


_mesh = plsc.VectorSubcoreMesh(core_axis_name="c", subcore_axis_name="s")


@functools.partial(
    pl.kernel,
    mesh=_mesh,
    compiler_params=pltpu.CompilerParams(use_tc_tiling_on_sc=False),
    out_type=(
        jax.ShapeDtypeStruct((_N_TXT // _CL, _CL, _DIM), jnp.float32),
        jax.ShapeDtypeStruct((_N_TOP // _CL, _CL, _DIM), jnp.float32),
    ),
    scratch_types=[
        pltpu.VMEM((_K, _CL), jnp.int32),
        pltpu.VMEM((_K, _CL, _DIM), jnp.float32),
        pltpu.SemaphoreType.DMA,
    ],
)
def _embed_lookup(text_hbm, topic_hbm, table_hbm, txt_out, top_out,
                  idx_v, rows_v, sem):
  wid = lax.axis_index("s") * _NC + lax.axis_index("c")
  _gather_stripe(text_hbm, txt_out, table_hbm, idx_v, rows_v, sem,
                 wid * _TXT_ROWS_W, _TXT_CHUNKS)
  _gather_stripe(topic_hbm, top_out, table_hbm, idx_v, rows_v, sem,
                 wid * _TOP_ROWS_W, _TOP_CHUNKS)


def kernel(text, topic, table):
  text2 = text.reshape(_N_TXT // _CL, _CL).astype(jnp.int32)
  topic2 = topic.reshape(_N_TOP // _CL, _CL).astype(jnp.int32)
  txt3, top3 = _embed_lookup(text2, topic2, table)
  return (txt3.reshape(_B, _L_TXT, _DIM), top3.reshape(_B, _L_TOP, _DIM))


# speedup vs baseline: 4.0404x; 1.0816x over previous
import functools

import jax
import jax.numpy as jnp
from jax import lax
from jax.experimental import pallas as pl
from jax.experimental.pallas import tpu as pltpu
from jax.experimental.pallas import tpu_sc as plsc

_VOCAB = 100000
_DIM = 64
_B = 4096
_L_TXT = 200
_L_TOP = 20
_N_TXT = _B * _L_TXT
_N_TOP = _B * _L_TOP

_NC = 2
_NS = 16
_NW = _NC * _NS

_CL = 128
_K = 4
_C = _K * _CL

_TXT_BLKS_W = _N_TXT // _NW // _CL
_TOP_BLKS_W = _N_TOP // _NW // _CL
_TXT_CHUNKS = _TXT_BLKS_W // _K
_TOP_CHUNKS = _TOP_BLKS_W // _K


def _pipelined_stripe(idx_hbm, out_hbm, table_hbm, idx_v, rows_v, sems,
                      base_blk, n_chunks):
  n = n_chunks

  def fire_idx(g, p):
    pltpu.async_copy(idx_hbm.at[pl.ds(base_blk + g * _K, _K)],
                     idx_v.at[p], sems[p][0])

  def fire_gathers(g, p):
    del g
    pltpu.make_async_copy(idx_hbm.at[pl.ds(base_blk, _K)], idx_v.at[p],
                          sems[p][0]).wait()
    for j in range(_K):
      pltpu.async_copy(table_hbm.at[idx_v.at[p, j]], rows_v.at[p, j],
                       sems[p][1])

  def fire_write(g, p):
    for j in range(_K):
      pltpu.make_async_copy(table_hbm.at[pl.ds(0, _CL)], rows_v.at[p, j],
                            sems[p][1]).wait()
    pltpu.async_copy(rows_v.at[p],
                     out_hbm.at[pl.ds(base_blk + g * _K, _K)], sems[p][2])

  def wait_write(g, p):
    pltpu.make_async_copy(rows_v.at[p],
                          out_hbm.at[pl.ds(base_blk + g * _K, _K)],
                          sems[p][2]).wait()

  def iter_block(g, p):
    q = 1 - p
    wait_write(g - 1, q)
    fire_gathers(g + 1, q)
    fire_write(g, p)
    @pl.when(g + 2 <= n - 1)
    def _():
      fire_idx(g + 2, p)

  fire_idx(0, 0)
  fire_idx(1, 1)
  fire_gathers(0, 0)
  if n >= 2:
    fire_gathers(1, 1)
  fire_write(0, 0)
  if n >= 3:
    fire_idx(2, 0)

  n_iter = max(n - 2, 0)
  n_pairs = n_iter // 2

  if n_pairs > 0:
    def pair_body(i, carry):
      g0 = 1 + 2 * i
      for b in range(2):
        iter_block(g0 + b, (1 + b) % 2)
      return carry

    lax.fori_loop(0, n_pairs, pair_body, 0)

  for g in range(1 + 2 * n_pairs, n - 1):
    iter_block(g, g % 2)

  if n >= 2:
    p_last = (n - 1) % 2
    wait_write(n - 2, 1 - p_last)
    fire_write(n - 1, p_last)
    wait_write(n - 1, p_last)
  else:
    wait_write(0, 0)


_mesh = plsc.VectorSubcoreMesh(core_axis_name="c", subcore_axis_name="s")


@functools.partial(
    pl.kernel,
    mesh=_mesh,
    compiler_params=pltpu.CompilerParams(use_tc_tiling_on_sc=False),
    out_type=(
        jax.ShapeDtypeStruct((_N_TXT // _CL, _CL, _DIM), jnp.float32),
        jax.ShapeDtypeStruct((_N_TOP // _CL, _CL, _DIM), jnp.float32),
    ),
    scratch_types=[
        pltpu.VMEM((2, _K, _CL), jnp.int32),
        pltpu.VMEM((2, _K, _CL, _DIM), jnp.float32),
        pltpu.SemaphoreType.DMA,
        pltpu.SemaphoreType.DMA,
        pltpu.SemaphoreType.DMA,
        pltpu.SemaphoreType.DMA,
        pltpu.SemaphoreType.DMA,
        pltpu.SemaphoreType.DMA,
    ],
)
def _embed_lookup(text_hbm, topic_hbm, table_hbm, txt_out, top_out,
                  idx_v, rows_v, si0, sg0, sw0, si1, sg1, sw1):
  wid = lax.axis_index("s") * _NC + lax.axis_index("c")
  sems = ((si0, sg0, sw0), (si1, sg1, sw1))
  _pipelined_stripe(text_hbm, txt_out, table_hbm, idx_v, rows_v, sems,
                    wid * _TXT_BLKS_W, _TXT_CHUNKS)
  _pipelined_stripe(topic_hbm, top_out, table_hbm, idx_v, rows_v, sems,
                    wid * _TOP_BLKS_W, _TOP_CHUNKS)


def kernel(text, topic, table):
  text2 = text.reshape(_N_TXT // _CL, _CL).astype(jnp.int32)
  topic2 = topic.reshape(_N_TOP // _CL, _CL).astype(jnp.int32)
  txt3, top3 = _embed_lookup(text2, topic2, table)
  return (txt3.reshape(_B, _L_TXT, _DIM), top3.reshape(_B, _L_TOP, _DIM))
